# baseline (speedup 1.0000x reference)
import functools

import jax
import jax.numpy as jnp
from jax import lax
from jax.experimental import pallas as pl
from jax.experimental.pallas import tpu as pltpu
from jax.experimental.pallas import tpu_sc as plsc

_VOCAB = 30522
_HIDDEN = 768
_EPS = 1e-12

_NC, _NS = 2, 16
_NW = _NC * _NS
_SEQ = 128
_BATCH = 1024
_TOK = _BATCH * _SEQ
_TPW = _TOK // _NW
_SPW = _TPW // _SEQ
_G = 16
_KPS = _SEQ // _G
_NBUF = 4

_NCH = 4
_TPC = _TOK // _NCH

_BT = 512


_mesh = plsc.VectorSubcoreMesh(core_axis_name="c", subcore_axis_name="s")


@functools.partial(
    pl.kernel,
    mesh=_mesh,
    compiler_params=pltpu.CompilerParams(needs_layout_passes=False),
    out_type=jax.ShapeDtypeStruct((_TOK // _NCH, _HIDDEN), jnp.float32),
    scratch_types=(
        [pltpu.VMEM((_SPW // _NCH, _KPS, _G), jnp.int32)]
        + [pltpu.VMEM((_G, _HIDDEN), jnp.float32) for _ in range(_NBUF)]
        + [pltpu.SemaphoreType.DMA for _ in range(2 * _NBUF)]
    ),
)
def _gather_kernel(word_hbm, idx_hbm, out_hbm, idx_l, *bufs_sems):
    nseq = _SPW // _NCH
    bufs = bufs_sems[:_NBUF]
    gsem = bufs_sems[_NBUF:2 * _NBUF]
    ssem = bufs_sems[2 * _NBUF:]
    wid = lax.axis_index("s") * _NC + lax.axis_index("c")

    pltpu.sync_copy(idx_hbm.at[wid], idx_l)

    for k in range(2):
        pltpu.async_copy(
            word_hbm.at[idx_l.at[0, k]], bufs[k], gsem[k])

    def seq_body(s, carry):
        for k in range(_KPS):
            k2 = (k + 2) % _NBUF
            pltpu.make_async_copy(
                word_hbm.at[pl.ds(0, _G)], bufs[k % _NBUF],
                gsem[k % _NBUF]).wait()
            obase = wid * (_TPW // _NCH) + s * _SEQ + k * _G
            pltpu.async_copy(
                bufs[k % _NBUF], out_hbm.at[pl.ds(obase, _G)],
                ssem[k % _NBUF])
            k2s = (k + 2) % _KPS
            if k < 2:
                @pl.when(s >= 1)
                def _():
                    pltpu.make_async_copy(
                        bufs[k2], out_hbm.at[pl.ds(0, _G)],
                        ssem[k2]).wait()
            else:
                pltpu.make_async_copy(
                    bufs[k2], out_hbm.at[pl.ds(0, _G)], ssem[k2]).wait()
            if k < _KPS - 2:
                pltpu.async_copy(
                    word_hbm.at[idx_l.at[s, k2s]], bufs[k2], gsem[k2])
            else:
                @pl.when(s < nseq - 1)
                def _():
                    pltpu.async_copy(
                        word_hbm.at[idx_l.at[s + 1, k2s]], bufs[k2],
                        gsem[k2])
        return carry

    lax.fori_loop(0, nseq, seq_body, 0)

    for k in (2, 3):
        pltpu.make_async_copy(
            bufs[k], out_hbm.at[pl.ds(0, _G)], ssem[k]).wait()


def _ln_body(x_ref, b_ref, g_ref, be_ref, o_ref):
    x = x_ref[...] + b_ref[...]
    m = jnp.mean(x, axis=1, keepdims=True)
    xc = x - m
    var = jnp.mean(xc * xc, axis=1, keepdims=True)
    y = xc * lax.rsqrt(var + _EPS)
    o_ref[...] = y * g_ref[...] + be_ref[...]


_ln_call = pl.pallas_call(
    _ln_body,
    grid=(_TPC // _BT,),
    in_specs=[
        pl.BlockSpec((_BT, _HIDDEN), lambda i: (i, 0)),
        pl.BlockSpec((_BT, _HIDDEN), lambda i: (0, 0)),
        pl.BlockSpec((1, _HIDDEN), lambda i: (0, 0)),
        pl.BlockSpec((1, _HIDDEN), lambda i: (0, 0)),
    ],
    out_specs=pl.BlockSpec((_BT, _HIDDEN), lambda i: (i, 0)),
    out_shape=jax.ShapeDtypeStruct((_TPC, _HIDDEN), jnp.float32),
)


def kernel(input_tokens, word_table, pos_table, type_table, ln_gamma, ln_beta):
    idx = input_tokens.astype(jnp.int32).reshape(
        _NCH, _NW, _SPW // _NCH, _KPS, _G)
    bias = jnp.tile(pos_table[:_SEQ] + type_table[0][None, :],
                    (_BT // _SEQ, 1))
    gam = ln_gamma.reshape(1, _HIDDEN)
    bet = ln_beta.reshape(1, _HIDDEN)
    gs = [_gather_kernel(word_table, idx[c]) for c in range(_NCH)]
    outs = [_ln_call(g, bias, gam, bet) for g in gs]
    out = jnp.concatenate(outs, axis=0)
    return out.reshape(_BATCH, _SEQ, _HIDDEN)


# speedup vs baseline: 1.4053x; 1.4053x over previous
import functools

import jax
import jax.numpy as jnp
from jax import lax
from jax.experimental import pallas as pl
from jax.experimental.pallas import tpu as pltpu
from jax.experimental.pallas import tpu_sc as plsc

_VOCAB = 30522
_HIDDEN = 768
_EPS = 1e-12

_NC, _NS = 2, 16
_NW = _NC * _NS
_SEQ = 128
_BATCH = 1024
_TOK = _BATCH * _SEQ
_TPW = _TOK // _NW
_SPW = _TPW // _SEQ
_G = 16
_KPS = _SEQ // _G
_NBUF = 4

_NCH = 1
_TPC = _TOK // _NCH

_BT = 512


_mesh = plsc.VectorSubcoreMesh(core_axis_name="c", subcore_axis_name="s")


@functools.partial(
    pl.kernel,
    mesh=_mesh,
    compiler_params=pltpu.CompilerParams(needs_layout_passes=False),
    out_type=jax.ShapeDtypeStruct((_TOK // _NCH, _HIDDEN), jnp.float32),
    scratch_types=(
        [pltpu.VMEM((_SPW // _NCH, _KPS, _G), jnp.int32)]
        + [pltpu.VMEM((_G, _HIDDEN), jnp.float32) for _ in range(_NBUF)]
        + [pltpu.SemaphoreType.DMA for _ in range(2 * _NBUF)]
    ),
)
def _gather_kernel(word_hbm, idx_hbm, out_hbm, idx_l, *bufs_sems):
    nseq = _SPW // _NCH
    bufs = bufs_sems[:_NBUF]
    gsem = bufs_sems[_NBUF:2 * _NBUF]
    ssem = bufs_sems[2 * _NBUF:]
    wid = lax.axis_index("s") * _NC + lax.axis_index("c")

    pltpu.sync_copy(idx_hbm.at[wid], idx_l)

    for k in range(2):
        pltpu.async_copy(
            word_hbm.at[idx_l.at[0, k]], bufs[k], gsem[k])

    def seq_body(s, carry):
        for k in range(_KPS):
            k2 = (k + 2) % _NBUF
            pltpu.make_async_copy(
                word_hbm.at[pl.ds(0, _G)], bufs[k % _NBUF],
                gsem[k % _NBUF]).wait()
            obase = wid * (_TPW // _NCH) + s * _SEQ + k * _G
            pltpu.async_copy(
                bufs[k % _NBUF], out_hbm.at[pl.ds(obase, _G)],
                ssem[k % _NBUF])
            k2s = (k + 2) % _KPS
            if k < 2:
                @pl.when(s >= 1)
                def _():
                    pltpu.make_async_copy(
                        bufs[k2], out_hbm.at[pl.ds(0, _G)],
                        ssem[k2]).wait()
            else:
                pltpu.make_async_copy(
                    bufs[k2], out_hbm.at[pl.ds(0, _G)], ssem[k2]).wait()
            if k < _KPS - 2:
                pltpu.async_copy(
                    word_hbm.at[idx_l.at[s, k2s]], bufs[k2], gsem[k2])
            else:
                @pl.when(s < nseq - 1)
                def _():
                    pltpu.async_copy(
                        word_hbm.at[idx_l.at[s + 1, k2s]], bufs[k2],
                        gsem[k2])
        return carry

    lax.fori_loop(0, nseq, seq_body, 0)

    for k in (2, 3):
        pltpu.make_async_copy(
            bufs[k], out_hbm.at[pl.ds(0, _G)], ssem[k]).wait()


def _ln_body(x_ref, b_ref, g_ref, be_ref, o_ref):
    x = x_ref[...] + b_ref[...]
    m = jnp.mean(x, axis=1, keepdims=True)
    xc = x - m
    var = jnp.mean(xc * xc, axis=1, keepdims=True)
    y = xc * lax.rsqrt(var + _EPS)
    o_ref[...] = y * g_ref[...] + be_ref[...]


_ln_call = pl.pallas_call(
    _ln_body,
    grid=(_TPC // _BT,),
    in_specs=[
        pl.BlockSpec((_BT, _HIDDEN), lambda i: (i, 0)),
        pl.BlockSpec((_BT, _HIDDEN), lambda i: (0, 0)),
        pl.BlockSpec((1, _HIDDEN), lambda i: (0, 0)),
        pl.BlockSpec((1, _HIDDEN), lambda i: (0, 0)),
    ],
    out_specs=pl.BlockSpec((_BT, _HIDDEN), lambda i: (i, 0)),
    out_shape=jax.ShapeDtypeStruct((_TPC, _HIDDEN), jnp.float32),
)


def kernel(input_tokens, word_table, pos_table, type_table, ln_gamma, ln_beta):
    idx = input_tokens.astype(jnp.int32).reshape(
        _NCH, _NW, _SPW // _NCH, _KPS, _G)
    bias = jnp.tile(pos_table[:_SEQ] + type_table[0][None, :],
                    (_BT // _SEQ, 1))
    gam = ln_gamma.reshape(1, _HIDDEN)
    bet = ln_beta.reshape(1, _HIDDEN)
    gs = [_gather_kernel(word_table, idx[c]) for c in range(_NCH)]
    outs = [_ln_call(g, bias, gam, bet) for g in gs]
    out = jnp.concatenate(outs, axis=0)
    return out.reshape(_BATCH, _SEQ, _HIDDEN)


# speedup vs baseline: 1.5782x; 1.1230x over previous
import functools

import jax
import jax.numpy as jnp
from jax import lax
from jax.experimental import pallas as pl
from jax.experimental.pallas import tpu as pltpu
from jax.experimental.pallas import tpu_sc as plsc

_VOCAB = 30522
_HIDDEN = 768
_EPS = 1e-12

_NC, _NS = 2, 16
_NW = _NC * _NS
_SEQ = 128
_BATCH = 1024
_TOK = _BATCH * _SEQ
_TPW = _TOK // _NW
_SPW = _TPW // _SEQ
_G = 16
_KPS = _SEQ // _G
_NBUF = 4

_NCH = 1
_TPC = _TOK // _NCH

_BT = 1024


_mesh = plsc.VectorSubcoreMesh(core_axis_name="c", subcore_axis_name="s")


@functools.partial(
    pl.kernel,
    mesh=_mesh,
    compiler_params=pltpu.CompilerParams(needs_layout_passes=False),
    out_type=jax.ShapeDtypeStruct((_TOK // _NCH, _HIDDEN), jnp.float32),
    scratch_types=(
        [pltpu.VMEM((_SPW // _NCH, _KPS, _G), jnp.int32)]
        + [pltpu.VMEM((_G, _HIDDEN), jnp.float32) for _ in range(_NBUF)]
        + [pltpu.SemaphoreType.DMA for _ in range(2 * _NBUF)]
    ),
)
def _gather_kernel(word_hbm, idx_hbm, out_hbm, idx_l, *bufs_sems):
    nseq = _SPW // _NCH
    bufs = bufs_sems[:_NBUF]
    gsem = bufs_sems[_NBUF:2 * _NBUF]
    ssem = bufs_sems[2 * _NBUF:]
    wid = lax.axis_index("s") * _NC + lax.axis_index("c")

    pltpu.sync_copy(idx_hbm.at[wid], idx_l)

    for k in range(2):
        pltpu.async_copy(
            word_hbm.at[idx_l.at[0, k]], bufs[k], gsem[k])

    def seq_body(s, carry):
        for k in range(_KPS):
            k2 = (k + 2) % _NBUF
            pltpu.make_async_copy(
                word_hbm.at[pl.ds(0, _G)], bufs[k % _NBUF],
                gsem[k % _NBUF]).wait()
            obase = wid * (_TPW // _NCH) + s * _SEQ + k * _G
            pltpu.async_copy(
                bufs[k % _NBUF], out_hbm.at[pl.ds(obase, _G)],
                ssem[k % _NBUF])
            k2s = (k + 2) % _KPS
            if k < 2:
                @pl.when(s >= 1)
                def _():
                    pltpu.make_async_copy(
                        bufs[k2], out_hbm.at[pl.ds(0, _G)],
                        ssem[k2]).wait()
            else:
                pltpu.make_async_copy(
                    bufs[k2], out_hbm.at[pl.ds(0, _G)], ssem[k2]).wait()
            if k < _KPS - 2:
                pltpu.async_copy(
                    word_hbm.at[idx_l.at[s, k2s]], bufs[k2], gsem[k2])
            else:
                @pl.when(s < nseq - 1)
                def _():
                    pltpu.async_copy(
                        word_hbm.at[idx_l.at[s + 1, k2s]], bufs[k2],
                        gsem[k2])
        return carry

    lax.fori_loop(0, nseq, seq_body, 0)

    for k in (2, 3):
        pltpu.make_async_copy(
            bufs[k], out_hbm.at[pl.ds(0, _G)], ssem[k]).wait()


def _ln_body(x_ref, b_ref, g_ref, be_ref, o_ref):
    x = x_ref[...] + b_ref[...]
    m = jnp.mean(x, axis=1, keepdims=True)
    xc = x - m
    var = jnp.mean(xc * xc, axis=1, keepdims=True)
    y = xc * lax.rsqrt(var + _EPS)
    o_ref[...] = y * g_ref[...] + be_ref[...]


_ln_call = pl.pallas_call(
    _ln_body,
    grid=(_TPC // _BT,),
    in_specs=[
        pl.BlockSpec((_BT, _HIDDEN), lambda i: (i, 0)),
        pl.BlockSpec((_BT, _HIDDEN), lambda i: (0, 0)),
        pl.BlockSpec((1, _HIDDEN), lambda i: (0, 0)),
        pl.BlockSpec((1, _HIDDEN), lambda i: (0, 0)),
    ],
    out_specs=pl.BlockSpec((_BT, _HIDDEN), lambda i: (i, 0)),
    out_shape=jax.ShapeDtypeStruct((_TPC, _HIDDEN), jnp.float32),
)


def kernel(input_tokens, word_table, pos_table, type_table, ln_gamma, ln_beta):
    idx = input_tokens.astype(jnp.int32).reshape(
        _NCH, _NW, _SPW // _NCH, _KPS, _G)
    bias = jnp.tile(pos_table[:_SEQ] + type_table[0][None, :],
                    (_BT // _SEQ, 1))
    gam = ln_gamma.reshape(1, _HIDDEN)
    bet = ln_beta.reshape(1, _HIDDEN)
    gs = [_gather_kernel(word_table, idx[c]) for c in range(_NCH)]
    outs = [_ln_call(g, bias, gam, bet) for g in gs]
    out = jnp.concatenate(outs, axis=0)
    return out.reshape(_BATCH, _SEQ, _HIDDEN)


# speedup vs baseline: 1.6367x; 1.0371x over previous
import functools

import jax
import jax.numpy as jnp
from jax import lax
from jax.experimental import pallas as pl
from jax.experimental.pallas import tpu as pltpu
from jax.experimental.pallas import tpu_sc as plsc

_VOCAB = 30522
_HIDDEN = 768
_EPS = 1e-12

_NC, _NS = 2, 16
_NW = _NC * _NS
_SEQ = 128
_BATCH = 1024
_TOK = _BATCH * _SEQ
_TPW = _TOK // _NW
_SPW = _TPW // _SEQ
_G = 16
_KPS = _SEQ // _G
_NBUF = 4

_NCH = 1
_TPC = _TOK // _NCH

_BT = 2048


_mesh = plsc.VectorSubcoreMesh(core_axis_name="c", subcore_axis_name="s")


@functools.partial(
    pl.kernel,
    mesh=_mesh,
    compiler_params=pltpu.CompilerParams(needs_layout_passes=False),
    out_type=jax.ShapeDtypeStruct((_TOK // _NCH, _HIDDEN), jnp.float32),
    scratch_types=(
        [pltpu.VMEM((_SPW // _NCH, _KPS, _G), jnp.int32)]
        + [pltpu.VMEM((_G, _HIDDEN), jnp.float32) for _ in range(_NBUF)]
        + [pltpu.SemaphoreType.DMA for _ in range(2 * _NBUF)]
    ),
)
def _gather_kernel(word_hbm, idx_hbm, out_hbm, idx_l, *bufs_sems):
    nseq = _SPW // _NCH
    bufs = bufs_sems[:_NBUF]
    gsem = bufs_sems[_NBUF:2 * _NBUF]
    ssem = bufs_sems[2 * _NBUF:]
    wid = lax.axis_index("s") * _NC + lax.axis_index("c")

    pltpu.sync_copy(idx_hbm.at[wid], idx_l)

    for k in range(2):
        pltpu.async_copy(
            word_hbm.at[idx_l.at[0, k]], bufs[k], gsem[k])

    def seq_body(s, carry):
        for k in range(_KPS):
            k2 = (k + 2) % _NBUF
            pltpu.make_async_copy(
                word_hbm.at[pl.ds(0, _G)], bufs[k % _NBUF],
                gsem[k % _NBUF]).wait()
            obase = wid * (_TPW // _NCH) + s * _SEQ + k * _G
            pltpu.async_copy(
                bufs[k % _NBUF], out_hbm.at[pl.ds(obase, _G)],
                ssem[k % _NBUF])
            k2s = (k + 2) % _KPS
            if k < 2:
                @pl.when(s >= 1)
                def _():
                    pltpu.make_async_copy(
                        bufs[k2], out_hbm.at[pl.ds(0, _G)],
                        ssem[k2]).wait()
            else:
                pltpu.make_async_copy(
                    bufs[k2], out_hbm.at[pl.ds(0, _G)], ssem[k2]).wait()
            if k < _KPS - 2:
                pltpu.async_copy(
                    word_hbm.at[idx_l.at[s, k2s]], bufs[k2], gsem[k2])
            else:
                @pl.when(s < nseq - 1)
                def _():
                    pltpu.async_copy(
                        word_hbm.at[idx_l.at[s + 1, k2s]], bufs[k2],
                        gsem[k2])
        return carry

    lax.fori_loop(0, nseq, seq_body, 0)

    for k in (2, 3):
        pltpu.make_async_copy(
            bufs[k], out_hbm.at[pl.ds(0, _G)], ssem[k]).wait()


def _ln_body(x_ref, b_ref, g_ref, be_ref, o_ref):
    x = x_ref[...] + b_ref[...]
    m = jnp.mean(x, axis=1, keepdims=True)
    xc = x - m
    var = jnp.mean(xc * xc, axis=1, keepdims=True)
    y = xc * lax.rsqrt(var + _EPS)
    o_ref[...] = y * g_ref[...] + be_ref[...]


_ln_call = pl.pallas_call(
    _ln_body,
    grid=(_TPC // _BT,),
    in_specs=[
        pl.BlockSpec((_BT, _HIDDEN), lambda i: (i, 0)),
        pl.BlockSpec((_BT, _HIDDEN), lambda i: (0, 0)),
        pl.BlockSpec((1, _HIDDEN), lambda i: (0, 0)),
        pl.BlockSpec((1, _HIDDEN), lambda i: (0, 0)),
    ],
    out_specs=pl.BlockSpec((_BT, _HIDDEN), lambda i: (i, 0)),
    out_shape=jax.ShapeDtypeStruct((_TPC, _HIDDEN), jnp.float32),
)


def kernel(input_tokens, word_table, pos_table, type_table, ln_gamma, ln_beta):
    idx = input_tokens.astype(jnp.int32).reshape(
        _NCH, _NW, _SPW // _NCH, _KPS, _G)
    bias = jnp.tile(pos_table[:_SEQ] + type_table[0][None, :],
                    (_BT // _SEQ, 1))
    gam = ln_gamma.reshape(1, _HIDDEN)
    bet = ln_beta.reshape(1, _HIDDEN)
    gs = [_gather_kernel(word_table, idx[c]) for c in range(_NCH)]
    outs = [_ln_call(g, bias, gam, bet) for g in gs]
    out = jnp.concatenate(outs, axis=0)
    return out.reshape(_BATCH, _SEQ, _HIDDEN)


# speedup vs baseline: 1.6529x; 1.0099x over previous
import functools

import jax
import jax.numpy as jnp
from jax import lax
from jax.experimental import pallas as pl
from jax.experimental.pallas import tpu as pltpu
from jax.experimental.pallas import tpu_sc as plsc

_VOCAB = 30522
_HIDDEN = 768
_EPS = 1e-12

_NC, _NS = 2, 16
_NW = _NC * _NS
_SEQ = 128
_BATCH = 1024
_TOK = _BATCH * _SEQ
_TPW = _TOK // _NW
_SPW = _TPW // _SEQ
_G = 16
_KPS = _SEQ // _G
_NBUF = 4

_NCH = 1
_TPC = _TOK // _NCH

_BT = 2048


_mesh = plsc.VectorSubcoreMesh(core_axis_name="c", subcore_axis_name="s")


@functools.partial(
    pl.kernel,
    mesh=_mesh,
    compiler_params=pltpu.CompilerParams(needs_layout_passes=False),
    out_type=jax.ShapeDtypeStruct((_TOK // _NCH, _HIDDEN), jnp.float32),
    scratch_types=(
        [pltpu.VMEM((_SPW // _NCH, _KPS, _G), jnp.int32)]
        + [pltpu.VMEM((_G, _HIDDEN), jnp.float32) for _ in range(_NBUF)]
        + [pltpu.SemaphoreType.DMA for _ in range(2 * _NBUF)]
    ),
)
def _gather_kernel(word_hbm, idx_hbm, out_hbm, idx_l, *bufs_sems):
    nseq = _SPW // _NCH
    bufs = bufs_sems[:_NBUF]
    gsem = bufs_sems[_NBUF:2 * _NBUF]
    ssem = bufs_sems[2 * _NBUF:]
    wid = lax.axis_index("s") * _NC + lax.axis_index("c")

    pltpu.sync_copy(idx_hbm.at[wid], idx_l)

    for k in range(2):
        pltpu.async_copy(
            word_hbm.at[idx_l.at[0, k]], bufs[k], gsem[k])

    def seq_body(s, carry):
        for k in range(_KPS):
            k2 = (k + 2) % _NBUF
            pltpu.make_async_copy(
                word_hbm.at[pl.ds(0, _G)], bufs[k % _NBUF],
                gsem[k % _NBUF]).wait()
            obase = wid * (_TPW // _NCH) + s * _SEQ + k * _G
            pltpu.async_copy(
                bufs[k % _NBUF], out_hbm.at[pl.ds(obase, _G)],
                ssem[k % _NBUF])
            k2s = (k + 2) % _KPS
            if k < 2:
                @pl.when(s >= 1)
                def _():
                    pltpu.make_async_copy(
                        bufs[k2], out_hbm.at[pl.ds(0, _G)],
                        ssem[k2]).wait()
            else:
                pltpu.make_async_copy(
                    bufs[k2], out_hbm.at[pl.ds(0, _G)], ssem[k2]).wait()
            if k < _KPS - 2:
                pltpu.async_copy(
                    word_hbm.at[idx_l.at[s, k2s]], bufs[k2], gsem[k2])
            else:
                @pl.when(s < nseq - 1)
                def _():
                    pltpu.async_copy(
                        word_hbm.at[idx_l.at[s + 1, k2s]], bufs[k2],
                        gsem[k2])
        return carry

    lax.fori_loop(0, nseq, seq_body, 0)

    for k in (2, 3):
        pltpu.make_async_copy(
            bufs[k], out_hbm.at[pl.ds(0, _G)], ssem[k]).wait()


def _ln_body(x_ref, b_ref, g_ref, be_ref, o_ref):
    x = x_ref[...].reshape(_BT // _SEQ, _SEQ, _HIDDEN) + b_ref[...][None]
    x = x.reshape(_BT, _HIDDEN)
    m = jnp.mean(x, axis=1, keepdims=True)
    xc = x - m
    var = jnp.mean(xc * xc, axis=1, keepdims=True)
    y = xc * lax.rsqrt(var + _EPS)
    o_ref[...] = y * g_ref[...] + be_ref[...]


_ln_call = pl.pallas_call(
    _ln_body,
    grid=(_TPC // _BT,),
    in_specs=[
        pl.BlockSpec((_BT, _HIDDEN), lambda i: (i, 0)),
        pl.BlockSpec((_SEQ, _HIDDEN), lambda i: (0, 0)),
        pl.BlockSpec((1, _HIDDEN), lambda i: (0, 0)),
        pl.BlockSpec((1, _HIDDEN), lambda i: (0, 0)),
    ],
    out_specs=pl.BlockSpec((_BT, _HIDDEN), lambda i: (i, 0)),
    out_shape=jax.ShapeDtypeStruct((_TPC, _HIDDEN), jnp.float32),
)


def kernel(input_tokens, word_table, pos_table, type_table, ln_gamma, ln_beta):
    idx = input_tokens.astype(jnp.int32).reshape(
        _NCH, _NW, _SPW // _NCH, _KPS, _G)
    bias = pos_table[:_SEQ] + type_table[0][None, :]
    gam = ln_gamma.reshape(1, _HIDDEN)
    bet = ln_beta.reshape(1, _HIDDEN)
    gs = [_gather_kernel(word_table, idx[c]) for c in range(_NCH)]
    outs = [_ln_call(g, bias, gam, bet) for g in gs]
    out = jnp.concatenate(outs, axis=0)
    return out.reshape(_BATCH, _SEQ, _HIDDEN)


# speedup vs baseline: 1.6571x; 1.0025x over previous
import functools

import jax
import jax.numpy as jnp
from jax import lax
from jax.experimental import pallas as pl
from jax.experimental.pallas import tpu as pltpu
from jax.experimental.pallas import tpu_sc as plsc

_VOCAB = 30522
_HIDDEN = 768
_EPS = 1e-12

_NC, _NS = 2, 16
_NW = _NC * _NS
_SEQ = 128
_BATCH = 1024
_TOK = _BATCH * _SEQ
_TPW = _TOK // _NW
_SPW = _TPW // _SEQ
_G = 32
_KPS = _SEQ // _G
_NBUF = 4

_NCH = 1
_TPC = _TOK // _NCH

_BT = 2048


_mesh = plsc.VectorSubcoreMesh(core_axis_name="c", subcore_axis_name="s")


@functools.partial(
    pl.kernel,
    mesh=_mesh,
    compiler_params=pltpu.CompilerParams(needs_layout_passes=False),
    out_type=jax.ShapeDtypeStruct((_TOK // _NCH, _HIDDEN), jnp.float32),
    scratch_types=(
        [pltpu.VMEM((_SPW // _NCH, _KPS, _G), jnp.int32)]
        + [pltpu.VMEM((_G, _HIDDEN), jnp.float32) for _ in range(_NBUF)]
        + [pltpu.SemaphoreType.DMA for _ in range(2 * _NBUF)]
    ),
)
def _gather_kernel(word_hbm, idx_hbm, out_hbm, idx_l, *bufs_sems):
    nseq = _SPW // _NCH
    bufs = bufs_sems[:_NBUF]
    gsem = bufs_sems[_NBUF:2 * _NBUF]
    ssem = bufs_sems[2 * _NBUF:]
    wid = lax.axis_index("s") * _NC + lax.axis_index("c")

    pltpu.sync_copy(idx_hbm.at[wid], idx_l)

    for k in range(2):
        pltpu.async_copy(
            word_hbm.at[idx_l.at[0, k]], bufs[k], gsem[k])

    def seq_body(s, carry):
        for k in range(_KPS):
            k2 = (k + 2) % _NBUF
            pltpu.make_async_copy(
                word_hbm.at[pl.ds(0, _G)], bufs[k % _NBUF],
                gsem[k % _NBUF]).wait()
            obase = wid * (_TPW // _NCH) + s * _SEQ + k * _G
            pltpu.async_copy(
                bufs[k % _NBUF], out_hbm.at[pl.ds(obase, _G)],
                ssem[k % _NBUF])
            k2s = (k + 2) % _KPS
            if k < 2:
                @pl.when(s >= 1)
                def _():
                    pltpu.make_async_copy(
                        bufs[k2], out_hbm.at[pl.ds(0, _G)],
                        ssem[k2]).wait()
            else:
                pltpu.make_async_copy(
                    bufs[k2], out_hbm.at[pl.ds(0, _G)], ssem[k2]).wait()
            if k < _KPS - 2:
                pltpu.async_copy(
                    word_hbm.at[idx_l.at[s, k2s]], bufs[k2], gsem[k2])
            else:
                @pl.when(s < nseq - 1)
                def _():
                    pltpu.async_copy(
                        word_hbm.at[idx_l.at[s + 1, k2s]], bufs[k2],
                        gsem[k2])
        return carry

    lax.fori_loop(0, nseq, seq_body, 0)

    for k in (2, 3):
        pltpu.make_async_copy(
            bufs[k], out_hbm.at[pl.ds(0, _G)], ssem[k]).wait()


def _ln_body(x_ref, b_ref, g_ref, be_ref, o_ref):
    x = x_ref[...].reshape(_BT // _SEQ, _SEQ, _HIDDEN) + b_ref[...][None]
    x = x.reshape(_BT, _HIDDEN)
    m = jnp.mean(x, axis=1, keepdims=True)
    xc = x - m
    var = jnp.mean(xc * xc, axis=1, keepdims=True)
    y = xc * lax.rsqrt(var + _EPS)
    o_ref[...] = y * g_ref[...] + be_ref[...]


_ln_call = pl.pallas_call(
    _ln_body,
    grid=(_TPC // _BT,),
    in_specs=[
        pl.BlockSpec((_BT, _HIDDEN), lambda i: (i, 0)),
        pl.BlockSpec((_SEQ, _HIDDEN), lambda i: (0, 0)),
        pl.BlockSpec((1, _HIDDEN), lambda i: (0, 0)),
        pl.BlockSpec((1, _HIDDEN), lambda i: (0, 0)),
    ],
    out_specs=pl.BlockSpec((_BT, _HIDDEN), lambda i: (i, 0)),
    out_shape=jax.ShapeDtypeStruct((_TPC, _HIDDEN), jnp.float32),
)


def kernel(input_tokens, word_table, pos_table, type_table, ln_gamma, ln_beta):
    idx = input_tokens.astype(jnp.int32).reshape(
        _NCH, _NW, _SPW // _NCH, _KPS, _G)
    bias = pos_table[:_SEQ] + type_table[0][None, :]
    gam = ln_gamma.reshape(1, _HIDDEN)
    bet = ln_beta.reshape(1, _HIDDEN)
    gs = [_gather_kernel(word_table, idx[c]) for c in range(_NCH)]
    outs = [_ln_call(g, bias, gam, bet) for g in gs]
    out = jnp.concatenate(outs, axis=0)
    return out.reshape(_BATCH, _SEQ, _HIDDEN)
